# baseline (device time: 441416 ns/iter reference)
import numpy as np

import jax
import jax.numpy as jnp
from jax import lax
from jax.experimental import pallas as pl
from jax.experimental.pallas import tpu as pltpu

N_DEV = 8
SQ = 2048
D = 1024
DH = 128
H_LOC = D // DH
QB = 512
CHUNK = SQ // N_DEV
SCALE = 0.08838834764831843

_inv = 1.0 / (10000.0 ** (np.arange(0, DH, 2) / DH))
_pos = np.arange(SQ)[:, None] * _inv[None, :]
_COS = jnp.asarray(np.repeat(np.cos(_pos), 2, axis=-1), dtype=jnp.float32)
_SIN = jnp.asarray(np.repeat(np.sin(_pos), 2, axis=-1), dtype=jnp.float32)


def _qkv_body(x_ref, wq_ref, wk_ref, wv_ref, q_ref, k_ref, v_ref):
    x = x_ref[...]
    q_ref[...] = jnp.dot(x, wq_ref[...], preferred_element_type=jnp.float32)
    k_ref[...] = jnp.dot(x, wk_ref[...], preferred_element_type=jnp.float32)
    v_ref[...] = jnp.dot(x, wv_ref[...], preferred_element_type=jnp.float32)


def _qkv(x2, Wq, Wk, Wv):
    w_spec = pl.BlockSpec((D, DH), lambda h: (0, h))
    o_spec = pl.BlockSpec((SQ, DH), lambda h: (0, h))
    o_shape = jax.ShapeDtypeStruct((SQ, D), jnp.float32)
    return pl.pallas_call(
        _qkv_body,
        grid=(H_LOC,),
        in_specs=[pl.BlockSpec((SQ, D), lambda h: (0, 0)), w_spec, w_spec, w_spec],
        out_specs=[o_spec, o_spec, o_spec],
        out_shape=[o_shape, o_shape, o_shape],
    )(x2, Wq, Wk, Wv)


def _rope(t):
    th = t.reshape(SQ, H_LOC, DH // 2, 2)
    tr = jnp.stack([-th[..., 1], th[..., 0]], axis=-1).reshape(SQ, H_LOC, DH)
    return (
        t.reshape(SQ, H_LOC, DH) * _COS[:, None, :] + tr * _SIN[:, None, :]
    ).reshape(SQ, D)


def _attn_body(q_ref, k_ref, v_ref, o_ref):
    s = lax.dot_general(
        q_ref[...], k_ref[...], (((1,), (1,)), ((), ())),
        preferred_element_type=jnp.float32,
    ) * SCALE
    m = jnp.max(s, axis=-1, keepdims=True)
    w = jnp.exp(s - m)
    w = w / jnp.sum(w, axis=-1, keepdims=True)
    o_ref[...] = jnp.dot(w, v_ref[...], preferred_element_type=jnp.float32)


def _attention(q, k, v):
    kv_spec = pl.BlockSpec((SQ, DH), lambda h, qb: (0, h))
    return pl.pallas_call(
        _attn_body,
        grid=(H_LOC, SQ // QB),
        in_specs=[
            pl.BlockSpec((QB, DH), lambda h, qb: (qb, h)),
            kv_spec,
            kv_spec,
        ],
        out_specs=pl.BlockSpec((QB, DH), lambda h, qb: (qb, h)),
        out_shape=jax.ShapeDtypeStruct((SQ, D), jnp.float32),
    )(q, k, v)


def _proj_ar_body(ctx_ref, wo_ref, o_ref, comm_ref, send_sems, recv_sems):
    me = lax.axis_index("i")
    left = lax.rem(me + N_DEV - 1, N_DEV)
    right = lax.rem(me + 1, N_DEV)

    wo = wo_ref[...]
    for c in range(N_DEV):
        o_ref[c] = jnp.dot(
            ctx_ref[pl.ds(c * CHUNK, CHUNK), :], wo,
            preferred_element_type=jnp.float32,
        )

    barrier = pltpu.get_barrier_semaphore()
    for nbr in (left, right):
        pl.semaphore_signal(
            barrier, inc=1, device_id=(nbr,),
            device_id_type=pl.DeviceIdType.MESH,
        )
    pl.semaphore_wait(barrier, 2)

    for h in range(N_DEV - 1):
        send_idx = lax.rem(me - h + N_DEV, N_DEV)
        recv_idx = lax.rem(me - h - 1 + N_DEV, N_DEV)
        rdma = pltpu.make_async_remote_copy(
            src_ref=o_ref.at[send_idx],
            dst_ref=comm_ref.at[h],
            send_sem=send_sems.at[h],
            recv_sem=recv_sems.at[h],
            device_id=(right,),
            device_id_type=pl.DeviceIdType.MESH,
        )
        rdma.start()
        rdma.wait()
        o_ref[recv_idx] = o_ref[recv_idx] + comm_ref[h]

    for h in range(N_DEV - 1):
        send_idx = lax.rem(me + 1 - h + N_DEV, N_DEV)
        rdma = pltpu.make_async_remote_copy(
            src_ref=o_ref.at[send_idx],
            dst_ref=o_ref.at[send_idx],
            send_sem=send_sems.at[N_DEV - 1 + h],
            recv_sem=recv_sems.at[N_DEV - 1 + h],
            device_id=(right,),
            device_id_type=pl.DeviceIdType.MESH,
        )
        rdma.start()
        rdma.wait()


def _proj_allreduce(ctx, Wo):
    n_sems = 2 * (N_DEV - 1)
    return pl.pallas_call(
        _proj_ar_body,
        in_specs=[
            pl.BlockSpec(memory_space=pltpu.VMEM),
            pl.BlockSpec(memory_space=pltpu.VMEM),
        ],
        out_specs=pl.BlockSpec(memory_space=pltpu.VMEM),
        out_shape=jax.ShapeDtypeStruct((N_DEV, CHUNK, D), jnp.float32),
        scratch_shapes=[
            pltpu.VMEM((N_DEV - 1, CHUNK, D), jnp.float32),
            pltpu.SemaphoreType.DMA((n_sems,)),
            pltpu.SemaphoreType.DMA((n_sems,)),
        ],
        compiler_params=pltpu.CompilerParams(collective_id=0),
    )(ctx, Wo)


def kernel(x, Wq, Wk, Wv, Wo):
    x2 = x.reshape(SQ, D)
    q, k, v = _qkv(x2, Wq, Wk, Wv)
    q = _rope(q)
    k = _rope(k)
    ctx = _attention(q, k, v)
    out = _proj_allreduce(ctx, Wo)
    return out.reshape(1, SQ, D)
